# baseline (speedup 1.0000x reference)
import functools

import jax
import jax.numpy as jnp
from jax import lax
from jax.experimental import pallas as pl
from jax.experimental.pallas import tpu as pltpu
from jax.experimental.pallas import tpu_sc as plsc

_NC = 2
_NS = 16
_L = 16
_H = 24
_BN = 4096
_BNL = 8192
_BN2 = 25600
_C = 10000


def _pad_sizes(n):
    npt = -(-n // _NS)
    npt = ((npt + _L - 1) // _L) * _L
    npad = npt * _NS
    npad = ((npad + _BN - 1) // _BN) * _BN
    npt = npad // _NS
    return npt, npad


def _fill_zero(ref, nwords):
    def body(i, carry):
        ref[pl.ds(i * _L, _L)] = jnp.zeros((_L,), jnp.float32)
        return carry

    lax.fori_loop(0, nwords // _L, body, 0)


def _build_indeg(n, e):
    nw = _NC * _NS
    ew = e // nw
    c = _C
    assert ew % c == 0 and e % nw == 0
    nch = ew // c
    assert nch % 2 == 0
    npt, npad = _pad_sizes(n)
    mesh = plsc.VectorSubcoreMesh(core_axis_name="c", subcore_axis_name="s",
                                  num_cores=_NC, num_subcores=_NS)

    @functools.partial(
        pl.kernel,
        out_type=jax.ShapeDtypeStruct((_NC * npad,), jnp.float32),
        mesh=mesh,
        scratch_types=[
            pltpu.VMEM((c,), jnp.int32),
            pltpu.VMEM((c,), jnp.int32),
            pltpu.VMEM((c,), jnp.float32),
            pltpu.VMEM((npt,), jnp.float32),
            pltpu.VMEM_SHARED((npad,), jnp.float32),
            pltpu.SemaphoreType.DMA,
            pltpu.SemaphoreType.DMA,
        ],
    )
    def indeg(ei_ref, out_ref, idx0, idx1, ones_v, zbuf_v, acc_sh, sem0, sem1):
        cid = lax.axis_index("c")
        sid = lax.axis_index("s")
        wid = sid * _NC + cid

        def fill_ones(i, carry):
            ones_v[pl.ds(i * _L, _L)] = jnp.ones((_L,), jnp.float32)
            return carry

        lax.fori_loop(0, c // _L, fill_ones, 0)
        _fill_zero(zbuf_v, npt)

        pltpu.sync_copy(zbuf_v, acc_sh.at[pl.ds(sid * npt, npt)])
        plsc.subcore_barrier()

        base = e + wid * ew
        bufs = (idx0, idx1)
        sems = (sem0, sem1)

        pltpu.async_copy(ei_ref.at[pl.ds(base, c)], idx0, sem0)

        def outer(jj, carry):
            for b in (0, 1):
                j = jj * 2 + b
                src = ei_ref.at[pl.ds(base + j * c, c)]
                pltpu.make_async_copy(src, bufs[b], sems[b]).wait()

                @pl.when(j + 1 < nch)
                def _():
                    nxt = ei_ref.at[pl.ds(base + (j + 1) * c, c)]
                    pltpu.async_copy(nxt, bufs[1 - b], sems[1 - b])

                pltpu.sync_copy(ones_v, acc_sh.at[bufs[b]], add=True)
            return carry

        lax.fori_loop(0, nch // 2, outer, 0)
        plsc.subcore_barrier()
        pltpu.sync_copy(acc_sh.at[pl.ds(sid * npt, npt)], zbuf_v)
        pltpu.sync_copy(zbuf_v, out_ref.at[pl.ds(cid * npad + sid * npt, npt)])

    return indeg


def _build_segsum(n, e):
    nw = _NC * _NS
    ew = e // nw
    c = _C
    assert ew % c == 0 and e % nw == 0
    nch = ew // c
    assert nch % 2 == 0
    npt, npad = _pad_sizes(n)
    mesh = plsc.VectorSubcoreMesh(core_axis_name="c", subcore_axis_name="s",
                                  num_cores=_NC, num_subcores=_NS)

    @functools.partial(
        pl.kernel,
        out_type=jax.ShapeDtypeStruct((_NC * npad,), jnp.float32),
        mesh=mesh,
        scratch_types=[
            pltpu.VMEM((c,), jnp.int32),
            pltpu.VMEM((c,), jnp.int32),
            pltpu.VMEM((c,), jnp.int32),
            pltpu.VMEM((c,), jnp.int32),
            pltpu.VMEM((c,), jnp.float32),
            pltpu.VMEM((c,), jnp.float32),
            pltpu.VMEM((npt,), jnp.float32),
            pltpu.VMEM_SHARED((npad,), jnp.float32),
            pltpu.VMEM_SHARED((npad,), jnp.float32),
            pltpu.SemaphoreType.DMA,
            pltpu.SemaphoreType.DMA,
            pltpu.SemaphoreType.DMA,
            pltpu.SemaphoreType.DMA,
            pltpu.SemaphoreType.DMA,
            pltpu.SemaphoreType.DMA,
        ],
    )
    def segsum(ei_ref, s_ref, out_ref, idxr0, idxr1, idxc0, idxc1, vals0,
               vals1, zbuf_v, s_sh, acc_sh, semr0, semr1, semc0, semc1,
               semg0, semg1):
        cid = lax.axis_index("c")
        sid = lax.axis_index("s")
        wid = sid * _NC + cid

        _fill_zero(zbuf_v, npt)
        pltpu.sync_copy(zbuf_v, acc_sh.at[pl.ds(sid * npt, npt)])
        pltpu.sync_copy(s_ref.at[pl.ds(sid * npt, npt)], zbuf_v)
        pltpu.sync_copy(zbuf_v, s_sh.at[pl.ds(sid * npt, npt)])
        plsc.subcore_barrier()

        base = wid * ew
        rbufs = (idxr0, idxr1)
        cbufs = (idxc0, idxc1)
        vbufs = (vals0, vals1)
        rsems = (semr0, semr1)
        csems = (semc0, semc1)
        gsems = (semg0, semg1)

        pltpu.async_copy(ei_ref.at[pl.ds(base, c)], idxr0, semr0)
        pltpu.async_copy(ei_ref.at[pl.ds(e + base, c)], idxc0, semc0)

        def outer(jj, carry):
            for b in (0, 1):
                j = jj * 2 + b
                rsrc = ei_ref.at[pl.ds(base + j * c, c)]
                csrc = ei_ref.at[pl.ds(e + base + j * c, c)]
                pltpu.make_async_copy(rsrc, rbufs[b], rsems[b]).wait()
                pltpu.make_async_copy(csrc, cbufs[b], csems[b]).wait()
                pltpu.async_copy(s_sh.at[rbufs[b]], vbufs[b], gsems[b])

                @pl.when(j > 0)
                def _():
                    pltpu.make_async_copy(s_sh.at[rbufs[1 - b]], vbufs[1 - b],
                                          gsems[1 - b]).wait()
                    pltpu.sync_copy(vbufs[1 - b], acc_sh.at[cbufs[1 - b]],
                                    add=True)

                @pl.when(j + 1 < nch)
                def _():
                    rn = ei_ref.at[pl.ds(base + (j + 1) * c, c)]
                    cn = ei_ref.at[pl.ds(e + base + (j + 1) * c, c)]
                    pltpu.async_copy(rn, rbufs[1 - b], rsems[1 - b])
                    pltpu.async_copy(cn, cbufs[1 - b], csems[1 - b])
            return carry

        lax.fori_loop(0, nch // 2, outer, 0)
        bl = (nch - 1) % 2
        pltpu.make_async_copy(s_sh.at[rbufs[bl]], vbufs[bl], gsems[bl]).wait()
        pltpu.sync_copy(vbufs[bl], acc_sh.at[cbufs[bl]], add=True)
        plsc.subcore_barrier()
        pltpu.sync_copy(acc_sh.at[pl.ds(sid * npt, npt)], zbuf_v)
        pltpu.sync_copy(zbuf_v, out_ref.at[pl.ds(cid * npad + sid * npt, npt)])

    return segsum


def _lstm_body(ht_ref, x_ref, whx_ref, wg_ref, h1_ref, c1_ref, z_ref):
    h0 = ht_ref[0, 0]
    c0 = ht_ref[1, 0]
    xv = x_ref[...]
    ones = jnp.ones((1, xv.shape[0]), jnp.float32)
    hx = jnp.concatenate([h0, xv[None, :], ones], axis=0)
    gates = lax.dot_general(whx_ref[...], hx, (((1,), (0,)), ((), ())),
                            preferred_element_type=jnp.float32)
    ig = jax.nn.sigmoid(gates[0:_H])
    fg = jax.nn.sigmoid(gates[_H:2 * _H])
    gg = jnp.tanh(gates[2 * _H:3 * _H])
    og = jax.nn.sigmoid(gates[3 * _H:4 * _H])
    c1 = fg * c0 + ig * gg
    h1 = og * jnp.tanh(c1)
    h1_ref[0] = h1
    c1_ref[0] = c1
    zm = lax.dot_general(wg_ref[...], h1, (((1,), (0,)), ((), ())),
                         preferred_element_type=jnp.float32)
    z_ref[...] = zm[0]


def _lstm_call(ht, x, whx, wg, n):
    grid = -(-n // _BNL)
    f32 = jnp.float32
    return pl.pallas_call(
        _lstm_body,
        grid=(grid,),
        in_specs=[
            pl.BlockSpec((2, 1, _H, _BNL), lambda i: (0, 0, 0, i)),
            pl.BlockSpec((_BNL,), lambda i: (i,)),
            pl.BlockSpec((4 * _H, _H + 2), lambda i: (0, 0)),
            pl.BlockSpec((1, _H), lambda i: (0, 0)),
        ],
        out_specs=[
            pl.BlockSpec((1, _H, _BNL), lambda i: (0, 0, i)),
            pl.BlockSpec((1, _H, _BNL), lambda i: (0, 0, i)),
            pl.BlockSpec((_BNL,), lambda i: (i,)),
        ],
        out_shape=[
            jax.ShapeDtypeStruct((1, _H, n), f32),
            jax.ShapeDtypeStruct((1, _H, n), f32),
            jax.ShapeDtypeStruct((n,), f32),
        ],
    )(ht, x, whx, wg)


def _scale_body(z_ref, dp0_ref, dp1_ref, dis_ref, s_ref):
    deg = dp0_ref[...] + dp1_ref[...] + 1.0
    dis = lax.rsqrt(deg)
    dis_ref[...] = dis
    s_ref[...] = dis * z_ref[...]


def _scale_call(z, degp, n, npad):
    grid = npad // _BN2
    off = npad // _BN2
    return pl.pallas_call(
        _scale_body,
        grid=(grid,),
        in_specs=[
            pl.BlockSpec((_BN2,), lambda i: (i,)),
            pl.BlockSpec((_BN2,), lambda i: (i,)),
            pl.BlockSpec((_BN2,), lambda i, off=off: (i + off,)),
        ],
        out_specs=[
            pl.BlockSpec((_BN2,), lambda i: (i,)),
            pl.BlockSpec((_BN2,), lambda i: (i,)),
        ],
        out_shape=[
            jax.ShapeDtypeStruct((n,), jnp.float32),
            jax.ShapeDtypeStruct((npad,), jnp.float32),
        ],
    )(z, degp, degp)


def _final_body(tp0_ref, tp1_ref, s_ref, dis_ref, x_ref, cc_ref, out_ref):
    t = tp0_ref[...] + tp1_ref[...] + s_ref[...]
    y = dis_ref[...] * t + cc_ref[0]
    out_ref[...] = x_ref[...] * y


def _final_call(tp, s, dis, x, cc, n, npad):
    grid = -(-n // _BN2)
    off = npad // _BN2
    return pl.pallas_call(
        _final_body,
        grid=(grid,),
        in_specs=[
            pl.BlockSpec((_BN2,), lambda i: (i,)),
            pl.BlockSpec((_BN2,), lambda i, off=off: (i + off,)),
            pl.BlockSpec((_BN2,), lambda i: (i,)),
            pl.BlockSpec((_BN2,), lambda i: (i,)),
            pl.BlockSpec((_BN2,), lambda i: (i,)),
            pl.BlockSpec(memory_space=pltpu.SMEM),
        ],
        out_specs=pl.BlockSpec((_BN2,), lambda i: (i,)),
        out_shape=jax.ShapeDtypeStruct((n,), jnp.float32),
    )(tp, tp, s, dis, x, cc)


@jax.jit
def kernel(x, h, edge_index, W_ih, W_hh, b_ih, b_hh, W_gcn, b_gcn, W_lin, b_lin):
    n = x.shape[0]
    e = edge_index.shape[1]
    npt, npad = _pad_sizes(n)

    w = W_lin[0]
    wg = (W_gcn @ w).reshape(1, _H)
    cc = (b_gcn @ w + b_lin[0]).reshape(1)
    whx = jnp.concatenate(
        [W_hh, W_ih, (b_ih + b_hh).reshape(4 * _H, 1)], axis=1)

    ht = jnp.transpose(h, (0, 1, 3, 2))
    ei_flat = edge_index.reshape(-1)
    degp = _build_indeg(n, e)(ei_flat)
    h1t, c1t, z = _lstm_call(ht, x, whx, wg, n)
    dis, s = _scale_call(z, degp, n, npad)
    tp = _build_segsum(n, e)(ei_flat, s)
    out0 = _final_call(tp, s, dis, x, cc, n, npad)
    return (out0, jnp.transpose(h1t, (0, 2, 1)), jnp.transpose(c1t, (0, 2, 1)))


# speedup vs baseline: 1.0291x; 1.0291x over previous
import functools

import jax
import jax.numpy as jnp
from jax import lax
from jax.experimental import pallas as pl
from jax.experimental.pallas import tpu as pltpu
from jax.experimental.pallas import tpu_sc as plsc

_NC = 2
_NS = 16
_L = 16
_H = 24
_BN = 4096
_BNL = 8192
_BN2 = 25600
_C = 10000


def _pad_sizes(n):
    npt = -(-n // _NS)
    npt = ((npt + _L - 1) // _L) * _L
    npad = npt * _NS
    npad = ((npad + _BN - 1) // _BN) * _BN
    npt = npad // _NS
    return npt, npad


def _fill_zero(ref, nwords):
    def body(i, carry):
        ref[pl.ds(i * _L, _L)] = jnp.zeros((_L,), jnp.float32)
        return carry

    lax.fori_loop(0, nwords // _L, body, 0)


def _build_indeg(n, e):
    nw = _NC * _NS
    ew = e // nw
    c = _C
    assert ew % c == 0 and e % nw == 0
    nch = ew // c
    assert nch % 2 == 0
    npt, npad = _pad_sizes(n)
    mesh = plsc.VectorSubcoreMesh(core_axis_name="c", subcore_axis_name="s",
                                  num_cores=_NC, num_subcores=_NS)

    @functools.partial(
        pl.kernel,
        out_type=jax.ShapeDtypeStruct((_NC * npad,), jnp.float32),
        mesh=mesh,
        scratch_types=[
            pltpu.VMEM((c,), jnp.int32),
            pltpu.VMEM((c,), jnp.int32),
            pltpu.VMEM((c,), jnp.float32),
            pltpu.VMEM((npt,), jnp.float32),
            pltpu.VMEM_SHARED((npad,), jnp.float32),
            pltpu.SemaphoreType.DMA,
            pltpu.SemaphoreType.DMA,
        ],
    )
    def indeg(ei_ref, out_ref, idx0, idx1, ones_v, zbuf_v, acc_sh, sem0, sem1):
        cid = lax.axis_index("c")
        sid = lax.axis_index("s")
        wid = sid * _NC + cid

        def fill_ones(i, carry):
            ones_v[pl.ds(i * _L, _L)] = jnp.ones((_L,), jnp.float32)
            return carry

        lax.fori_loop(0, c // _L, fill_ones, 0)
        _fill_zero(zbuf_v, npt)

        pltpu.sync_copy(zbuf_v, acc_sh.at[pl.ds(sid * npt, npt)])
        plsc.subcore_barrier()

        base = e + wid * ew
        bufs = (idx0, idx1)
        sems = (sem0, sem1)

        pltpu.async_copy(ei_ref.at[pl.ds(base, c)], idx0, sem0)

        def outer(jj, carry):
            for b in (0, 1):
                j = jj * 2 + b
                src = ei_ref.at[pl.ds(base + j * c, c)]
                pltpu.make_async_copy(src, bufs[b], sems[b]).wait()

                @pl.when(j + 1 < nch)
                def _():
                    nxt = ei_ref.at[pl.ds(base + (j + 1) * c, c)]
                    pltpu.async_copy(nxt, bufs[1 - b], sems[1 - b])

                pltpu.sync_copy(ones_v, acc_sh.at[bufs[b]], add=True)
            return carry

        lax.fori_loop(0, nch // 2, outer, 0)
        plsc.subcore_barrier()
        pltpu.sync_copy(acc_sh.at[pl.ds(sid * npt, npt)], zbuf_v)
        pltpu.sync_copy(zbuf_v, out_ref.at[pl.ds(cid * npad + sid * npt, npt)])

    return indeg


def _build_segsum(n, e):
    nw = _NC * _NS
    ew = e // nw
    c = _C
    assert ew % c == 0 and e % nw == 0
    nch = ew // c
    assert nch % 2 == 0
    npt, npad = _pad_sizes(n)
    mesh = plsc.VectorSubcoreMesh(core_axis_name="c", subcore_axis_name="s",
                                  num_cores=_NC, num_subcores=_NS)

    @functools.partial(
        pl.kernel,
        out_type=jax.ShapeDtypeStruct((_NC * npad,), jnp.float32),
        mesh=mesh,
        scratch_types=[
            pltpu.VMEM((c,), jnp.int32),
            pltpu.VMEM((c,), jnp.int32),
            pltpu.VMEM((c,), jnp.int32),
            pltpu.VMEM((c,), jnp.int32),
            pltpu.VMEM((c,), jnp.float32),
            pltpu.VMEM((npt,), jnp.float32),
            pltpu.VMEM_SHARED((npad,), jnp.float32),
            pltpu.VMEM_SHARED((npad,), jnp.float32),
            pltpu.SemaphoreType.DMA,
            pltpu.SemaphoreType.DMA,
            pltpu.SemaphoreType.DMA,
            pltpu.SemaphoreType.DMA,
        ],
    )
    def segsum(ei_ref, s_ref, out_ref, idxr0, idxr1, idxc0, idxc1, vals_v,
               zbuf_v, s_sh, acc_sh, semr0, semr1, semc0, semc1):
        cid = lax.axis_index("c")
        sid = lax.axis_index("s")
        wid = sid * _NC + cid

        _fill_zero(zbuf_v, npt)
        pltpu.sync_copy(zbuf_v, acc_sh.at[pl.ds(sid * npt, npt)])
        pltpu.sync_copy(s_ref.at[pl.ds(sid * npt, npt)], zbuf_v)
        pltpu.sync_copy(zbuf_v, s_sh.at[pl.ds(sid * npt, npt)])
        plsc.subcore_barrier()

        base = wid * ew
        rbufs = (idxr0, idxr1)
        cbufs = (idxc0, idxc1)
        rsems = (semr0, semr1)
        csems = (semc0, semc1)

        pltpu.async_copy(ei_ref.at[pl.ds(base, c)], idxr0, semr0)
        pltpu.async_copy(ei_ref.at[pl.ds(e + base, c)], idxc0, semc0)

        def outer(jj, carry):
            for b in (0, 1):
                j = jj * 2 + b
                rsrc = ei_ref.at[pl.ds(base + j * c, c)]
                csrc = ei_ref.at[pl.ds(e + base + j * c, c)]
                pltpu.make_async_copy(rsrc, rbufs[b], rsems[b]).wait()
                pltpu.make_async_copy(csrc, cbufs[b], csems[b]).wait()

                @pl.when(j + 1 < nch)
                def _():
                    rn = ei_ref.at[pl.ds(base + (j + 1) * c, c)]
                    cn = ei_ref.at[pl.ds(e + base + (j + 1) * c, c)]
                    pltpu.async_copy(rn, rbufs[1 - b], rsems[1 - b])
                    pltpu.async_copy(cn, cbufs[1 - b], csems[1 - b])

                pltpu.sync_copy(s_sh.at[rbufs[b]], vals_v)
                pltpu.sync_copy(vals_v, acc_sh.at[cbufs[b]], add=True)
            return carry

        lax.fori_loop(0, nch // 2, outer, 0)
        plsc.subcore_barrier()
        pltpu.sync_copy(acc_sh.at[pl.ds(sid * npt, npt)], zbuf_v)
        pltpu.sync_copy(zbuf_v, out_ref.at[pl.ds(cid * npad + sid * npt, npt)])

    return segsum


def _lstm_body(ht_ref, x_ref, whx_ref, wg_ref, h1_ref, c1_ref, z_ref):
    h0 = ht_ref[0, 0]
    c0 = ht_ref[1, 0]
    xv = x_ref[...]
    ones = jnp.ones((1, xv.shape[0]), jnp.float32)
    hx = jnp.concatenate([h0, xv[None, :], ones], axis=0)
    gates = lax.dot_general(whx_ref[...], hx, (((1,), (0,)), ((), ())),
                            preferred_element_type=jnp.float32)
    ig = jax.nn.sigmoid(gates[0:_H])
    fg = jax.nn.sigmoid(gates[_H:2 * _H])
    gg = jnp.tanh(gates[2 * _H:3 * _H])
    og = jax.nn.sigmoid(gates[3 * _H:4 * _H])
    c1 = fg * c0 + ig * gg
    h1 = og * jnp.tanh(c1)
    h1_ref[0] = h1
    c1_ref[0] = c1
    zm = lax.dot_general(wg_ref[...], h1, (((1,), (0,)), ((), ())),
                         preferred_element_type=jnp.float32)
    z_ref[...] = zm[0]


def _lstm_call(ht, x, whx, wg, n):
    grid = -(-n // _BNL)
    f32 = jnp.float32
    return pl.pallas_call(
        _lstm_body,
        grid=(grid,),
        in_specs=[
            pl.BlockSpec((2, 1, _H, _BNL), lambda i: (0, 0, 0, i)),
            pl.BlockSpec((_BNL,), lambda i: (i,)),
            pl.BlockSpec((4 * _H, _H + 2), lambda i: (0, 0)),
            pl.BlockSpec((1, _H), lambda i: (0, 0)),
        ],
        out_specs=[
            pl.BlockSpec((1, _H, _BNL), lambda i: (0, 0, i)),
            pl.BlockSpec((1, _H, _BNL), lambda i: (0, 0, i)),
            pl.BlockSpec((_BNL,), lambda i: (i,)),
        ],
        out_shape=[
            jax.ShapeDtypeStruct((1, _H, n), f32),
            jax.ShapeDtypeStruct((1, _H, n), f32),
            jax.ShapeDtypeStruct((n,), f32),
        ],
    )(ht, x, whx, wg)


def _scale_body(z_ref, dp0_ref, dp1_ref, dis_ref, s_ref):
    deg = dp0_ref[...] + dp1_ref[...] + 1.0
    dis = lax.rsqrt(deg)
    dis_ref[...] = dis
    s_ref[...] = dis * z_ref[...]


def _scale_call(z, degp, n, npad):
    grid = npad // _BN2
    off = npad // _BN2
    return pl.pallas_call(
        _scale_body,
        grid=(grid,),
        in_specs=[
            pl.BlockSpec((_BN2,), lambda i: (i,)),
            pl.BlockSpec((_BN2,), lambda i: (i,)),
            pl.BlockSpec((_BN2,), lambda i, off=off: (i + off,)),
        ],
        out_specs=[
            pl.BlockSpec((_BN2,), lambda i: (i,)),
            pl.BlockSpec((_BN2,), lambda i: (i,)),
        ],
        out_shape=[
            jax.ShapeDtypeStruct((n,), jnp.float32),
            jax.ShapeDtypeStruct((npad,), jnp.float32),
        ],
    )(z, degp, degp)


def _final_body(tp0_ref, tp1_ref, s_ref, dis_ref, x_ref, cc_ref, out_ref):
    t = tp0_ref[...] + tp1_ref[...] + s_ref[...]
    y = dis_ref[...] * t + cc_ref[0]
    out_ref[...] = x_ref[...] * y


def _final_call(tp, s, dis, x, cc, n, npad):
    grid = -(-n // _BN2)
    off = npad // _BN2
    return pl.pallas_call(
        _final_body,
        grid=(grid,),
        in_specs=[
            pl.BlockSpec((_BN2,), lambda i: (i,)),
            pl.BlockSpec((_BN2,), lambda i, off=off: (i + off,)),
            pl.BlockSpec((_BN2,), lambda i: (i,)),
            pl.BlockSpec((_BN2,), lambda i: (i,)),
            pl.BlockSpec((_BN2,), lambda i: (i,)),
            pl.BlockSpec(memory_space=pltpu.SMEM),
        ],
        out_specs=pl.BlockSpec((_BN2,), lambda i: (i,)),
        out_shape=jax.ShapeDtypeStruct((n,), jnp.float32),
    )(tp, tp, s, dis, x, cc)


@jax.jit
def kernel(x, h, edge_index, W_ih, W_hh, b_ih, b_hh, W_gcn, b_gcn, W_lin, b_lin):
    n = x.shape[0]
    e = edge_index.shape[1]
    npt, npad = _pad_sizes(n)

    w = W_lin[0]
    wg = (W_gcn @ w).reshape(1, _H)
    cc = (b_gcn @ w + b_lin[0]).reshape(1)
    whx = jnp.concatenate(
        [W_hh, W_ih, (b_ih + b_hh).reshape(4 * _H, 1)], axis=1)

    ht = jnp.transpose(h, (0, 1, 3, 2))
    ei_flat = edge_index.reshape(-1)
    degp = _build_indeg(n, e)(ei_flat)
    h1t, c1t, z = _lstm_call(ht, x, whx, wg, n)
    dis, s = _scale_call(z, degp, n, npad)
    tp = _build_segsum(n, e)(ei_flat, s)
    out0 = _final_call(tp, s, dis, x, cc, n, npad)
    return (out0, jnp.transpose(h1t, (0, 2, 1)), jnp.transpose(c1t, (0, 2, 1)))
